# baseline (speedup 1.0000x reference)
import functools

import jax
import jax.numpy as jnp
from jax import lax
from jax.experimental import pallas as pl
from jax.experimental.pallas import tpu as pltpu
from jax.experimental.pallas import tpu_sc as plsc

N_NODES = 10000
N_EDGES = 320000
D_FEAT = 128

NC = 2
NS = 16
NW = NC * NS

EPW = N_EDGES // NW
CHUNK = 80
NCHUNKS = EPW // CHUNK
NBUF = 3
NMAIN = NCHUNKS - (NCHUNKS % NBUF)

ZN = 8
OROWS = 624
OTAIL = N_NODES - OROWS * NS


def _sc_scatter_gather(x, src3, dst3):
  mesh = plsc.VectorSubcoreMesh(core_axis_name="c", subcore_axis_name="s")

  @functools.partial(
      pl.kernel,
      out_type=jax.ShapeDtypeStruct((NC, N_NODES, D_FEAT), jnp.float32),
      mesh=mesh,
      scratch_types=[
          pltpu.VMEM((NCHUNKS, CHUNK), jnp.int32),
          [pltpu.VMEM((CHUNK,), jnp.int32)] * NBUF,
          [pltpu.VMEM((CHUNK, D_FEAT), jnp.float32)] * NBUF,
          pltpu.VMEM_SHARED((N_NODES, D_FEAT), jnp.float32),
          [pltpu.SemaphoreType.DMA] * NBUF,
          [pltpu.SemaphoreType.DMA] * NBUF,
          [pltpu.SemaphoreType.DMA] * NBUF,
      ],
  )
  def k(x_hbm, src_hbm, dst_hbm, out_hbm,
        srcs, dstv, rows, acc, gsem, dsem, ssem):
    cid = lax.axis_index("c")
    sid = lax.axis_index("s")
    tile = cid * NS + sid

    pltpu.sync_copy(src_hbm.at[tile], srcs)
    ebase = tile * EPW

    @pl.loop(0, CHUNK)
    def _zrow(i):
      for j in range(D_FEAT // 16):
        rows[0][i, pl.ds(j * 16, 16)] = jnp.zeros((16,), jnp.float32)

    @pl.loop(0, ZN)
    def _zacc(z):
      pltpu.sync_copy(rows[0], acc.at[pl.ds(sid * OROWS + z * CHUNK, CHUNK)])

    plsc.subcore_barrier()

    for c in range(NBUF - 1):
      pltpu.async_copy(dst_hbm.at[pl.ds(ebase + c * CHUNK, CHUNK)],
                       dstv[c], dsem[c])
      pltpu.async_copy(x_hbm.at[srcs.at[c]], rows[c], gsem[c])

    def _do_chunk(ci, b, bo):
      pltpu.make_async_copy(x_hbm.at[srcs.at[ci]], rows[b], gsem[b]).wait()
      pltpu.make_async_copy(dst_hbm.at[pl.ds(0, CHUNK)], dstv[b],
                            dsem[b]).wait()
      pltpu.async_copy(rows[b], acc.at[dstv[b]], ssem[b], add=True)

      @pl.when(ci > 0)
      def _drain_prev():
        pltpu.make_async_copy(rows[bo], acc.at[dstv[bo]], ssem[bo]).wait()

      @pl.when(ci + NBUF - 1 < NCHUNKS)
      def _prefetch():
        nc = ci + NBUF - 1
        pltpu.async_copy(dst_hbm.at[pl.ds(ebase + nc * CHUNK, CHUNK)],
                         dstv[bo], dsem[bo])
        pltpu.async_copy(x_hbm.at[srcs.at[nc]], rows[bo], gsem[bo])

    @pl.loop(0, NMAIN, step=NBUF)
    def _group(ci0):
      for b in range(NBUF):
        _do_chunk(ci0 + b, b, (b + NBUF - 1) % NBUF)

    for ci in range(NMAIN, NCHUNKS):
      _do_chunk(ci, ci % NBUF, (ci + NBUF - 1) % NBUF)

    bl = (NCHUNKS - 1) % NBUF
    pltpu.make_async_copy(rows[bl], acc.at[dstv[bl]], ssem[bl]).wait()

    plsc.subcore_barrier()
    pltpu.sync_copy(acc.at[pl.ds(sid * OROWS, OROWS)],
                    out_hbm.at[cid, pl.ds(sid * OROWS, OROWS)])

    @pl.when(sid == NS - 1)
    def _tail():
      pltpu.sync_copy(acc.at[pl.ds(OROWS * NS, OTAIL)],
                      out_hbm.at[cid, pl.ds(OROWS * NS, OTAIL)])

  return k(x, src3, dst3)


def _tc_add(a, b):
  def body(a_ref, b_ref, o_ref):
    o_ref[...] = a_ref[...] + b_ref[...]

  blk = 1000
  return pl.pallas_call(
      body,
      out_shape=jax.ShapeDtypeStruct((N_NODES, D_FEAT), jnp.float32),
      grid=(N_NODES // blk,),
      in_specs=[
          pl.BlockSpec((blk, D_FEAT), lambda i: (i, 0)),
          pl.BlockSpec((blk, D_FEAT), lambda i: (i, 0)),
      ],
      out_specs=pl.BlockSpec((blk, D_FEAT), lambda i: (i, 0)),
  )(a, b)


@jax.jit
def kernel(x, edge_index):
  src3 = edge_index[0].reshape(NW, NCHUNKS, CHUNK)
  partial = _sc_scatter_gather(x, src3, edge_index[1])
  return _tc_add(partial[0], partial[1])


# speedup vs baseline: 1.0529x; 1.0529x over previous
import functools

import jax
import jax.numpy as jnp
from jax import lax
from jax.experimental import pallas as pl
from jax.experimental.pallas import tpu as pltpu
from jax.experimental.pallas import tpu_sc as plsc

N_NODES = 10000
N_EDGES = 320000
D_FEAT = 128

NC = 2
NS = 16
NW = NC * NS

EPW = N_EDGES // NW
CHUNK = 80
NCHUNKS = EPW // CHUNK
NBUF = 4
NMAIN = NCHUNKS - (NCHUNKS % NBUF)

ZN = 8
OROWS = 624
OTAIL = N_NODES - OROWS * NS


def _sc_scatter_gather(x, src, dst):
  mesh = plsc.VectorSubcoreMesh(core_axis_name="c", subcore_axis_name="s")

  @functools.partial(
      pl.kernel,
      out_type=jax.ShapeDtypeStruct((NC, N_NODES, D_FEAT), jnp.float32),
      mesh=mesh,
      scratch_types=[
          [pltpu.VMEM((CHUNK,), jnp.int32)] * NBUF,
          [pltpu.VMEM((CHUNK,), jnp.int32)] * NBUF,
          [pltpu.VMEM((CHUNK, D_FEAT), jnp.float32)] * NBUF,
          pltpu.VMEM_SHARED((N_NODES, D_FEAT), jnp.float32),
          [pltpu.SemaphoreType.DMA] * NBUF,
          [pltpu.SemaphoreType.DMA] * NBUF,
          [pltpu.SemaphoreType.DMA] * NBUF,
          [pltpu.SemaphoreType.DMA] * NBUF,
      ],
  )
  def k(x_hbm, src_hbm, dst_hbm, out_hbm,
        srcv, dstv, rows, acc, isem, dsem, gsem, ssem):
    cid = lax.axis_index("c")
    sid = lax.axis_index("s")
    tile = cid * NS + sid
    ebase = tile * EPW

    @pl.loop(0, CHUNK)
    def _zrow(i):
      for j in range(D_FEAT // 16):
        rows[0][i, pl.ds(j * 16, 16)] = jnp.zeros((16,), jnp.float32)

    @pl.loop(0, ZN)
    def _zacc(z):
      pltpu.sync_copy(rows[0], acc.at[pl.ds(sid * OROWS + z * CHUNK, CHUNK)])

    plsc.subcore_barrier()

    def load_idx(c, b):
      pltpu.async_copy(src_hbm.at[pl.ds(ebase + c * CHUNK, CHUNK)],
                       srcv[b], isem[b])
      pltpu.async_copy(dst_hbm.at[pl.ds(ebase + c * CHUNK, CHUNK)],
                       dstv[b], dsem[b])

    def gather(c, b):
      pltpu.make_async_copy(src_hbm.at[pl.ds(0, CHUNK)], srcv[b],
                            isem[b]).wait()
      pltpu.async_copy(x_hbm.at[srcv[b]], rows[b], gsem[b])

    for c in range(NBUF):
      load_idx(c, c)
    for c in range(NBUF - 1):
      gather(c, c)

    def _do_chunk(ci, b, bo):
      pltpu.make_async_copy(x_hbm.at[srcv[b]], rows[b], gsem[b]).wait()
      pltpu.make_async_copy(dst_hbm.at[pl.ds(0, CHUNK)], dstv[b],
                            dsem[b]).wait()
      pltpu.async_copy(rows[b], acc.at[dstv[b]], ssem[b], add=True).wait()

      @pl.when(ci + NBUF < NCHUNKS)
      def _prefetch_idx():
        load_idx(ci + NBUF, b)

      @pl.when(ci + NBUF - 1 < NCHUNKS)
      def _prefetch_gather():
        gather(ci + NBUF - 1, bo)

    @pl.loop(0, NMAIN, step=NBUF)
    def _group(ci0):
      for b in range(NBUF):
        _do_chunk(ci0 + b, b, (b + NBUF - 1) % NBUF)

    for ci in range(NMAIN, NCHUNKS):
      _do_chunk(ci, ci % NBUF, (ci + NBUF - 1) % NBUF)

    plsc.subcore_barrier()
    pltpu.sync_copy(acc.at[pl.ds(sid * OROWS, OROWS)],
                    out_hbm.at[cid, pl.ds(sid * OROWS, OROWS)])

    @pl.when(sid == NS - 1)
    def _tail():
      pltpu.sync_copy(acc.at[pl.ds(OROWS * NS, OTAIL)],
                      out_hbm.at[cid, pl.ds(OROWS * NS, OTAIL)])

  return k(x, src, dst)


def _tc_add(a, b):
  def body(a_ref, b_ref, o_ref):
    o_ref[...] = a_ref[...] + b_ref[...]

  blk = 1000
  return pl.pallas_call(
      body,
      out_shape=jax.ShapeDtypeStruct((N_NODES, D_FEAT), jnp.float32),
      grid=(N_NODES // blk,),
      in_specs=[
          pl.BlockSpec((blk, D_FEAT), lambda i: (i, 0)),
          pl.BlockSpec((blk, D_FEAT), lambda i: (i, 0)),
      ],
      out_specs=pl.BlockSpec((blk, D_FEAT), lambda i: (i, 0)),
  )(a, b)


@jax.jit
def kernel(x, edge_index):
  partial = _sc_scatter_gather(x, edge_index[0], edge_index[1])
  return _tc_add(partial[0], partial[1])


# speedup vs baseline: 1.2004x; 1.1401x over previous
import functools

import jax
import jax.numpy as jnp
from jax import lax
from jax.experimental import pallas as pl
from jax.experimental.pallas import tpu as pltpu
from jax.experimental.pallas import tpu_sc as plsc

N_NODES = 10000
N_EDGES = 320000
D_FEAT = 128

NC = 2
NS = 16
NW = NC * NS

EPW = N_EDGES // NW
CHUNK = 80
NCHUNKS = EPW // CHUNK
NBUF = 4
NMAIN = NCHUNKS - (NCHUNKS % NBUF)

ZN = 8
OROWS = 624
OTAIL = N_NODES - OROWS * NS


def _sc_scatter_gather(x, edge_index):
  mesh = plsc.VectorSubcoreMesh(core_axis_name="c", subcore_axis_name="s")

  @functools.partial(
      pl.kernel,
      out_type=jax.ShapeDtypeStruct((NC, N_NODES, D_FEAT), jnp.float32),
      mesh=mesh,
      scratch_types=[
          [pltpu.VMEM((CHUNK,), jnp.int32)] * NBUF,
          [pltpu.VMEM((CHUNK,), jnp.int32)] * NBUF,
          [pltpu.VMEM((CHUNK, D_FEAT), jnp.float32)] * NBUF,
          pltpu.VMEM_SHARED((N_NODES, D_FEAT), jnp.float32),
          [pltpu.SemaphoreType.DMA] * NBUF,
          [pltpu.SemaphoreType.DMA] * NBUF,
          [pltpu.SemaphoreType.DMA] * NBUF,
          [pltpu.SemaphoreType.DMA] * NBUF,
      ],
  )
  def k(x_hbm, ei_hbm, out_hbm,
        srcv, dstv, rows, acc, isem, dsem, gsem, ssem):
    cid = lax.axis_index("c")
    sid = lax.axis_index("s")
    tile = cid * NS + sid
    ebase = tile * EPW

    @pl.loop(0, CHUNK)
    def _zrow(i):
      for j in range(D_FEAT // 16):
        rows[0][i, pl.ds(j * 16, 16)] = jnp.zeros((16,), jnp.float32)

    @pl.loop(0, ZN)
    def _zacc(z):
      pltpu.sync_copy(rows[0], acc.at[pl.ds(sid * OROWS + z * CHUNK, CHUNK)])

    plsc.subcore_barrier()

    def load_idx(c, b):
      pltpu.async_copy(ei_hbm.at[pl.ds(ebase + c * CHUNK, CHUNK)],
                       srcv[b], isem[b])
      pltpu.async_copy(ei_hbm.at[pl.ds(N_EDGES + ebase + c * CHUNK, CHUNK)],
                       dstv[b], dsem[b])

    def gather(c, b):
      pltpu.make_async_copy(ei_hbm.at[pl.ds(0, CHUNK)], srcv[b],
                            isem[b]).wait()
      pltpu.async_copy(x_hbm.at[srcv[b]], rows[b], gsem[b])

    for c in range(NBUF):
      load_idx(c, c)
    for c in range(NBUF - 1):
      gather(c, c)

    def _do_chunk(ci, b, bo):
      pltpu.make_async_copy(x_hbm.at[srcv[b]], rows[b], gsem[b]).wait()
      pltpu.make_async_copy(ei_hbm.at[pl.ds(0, CHUNK)], dstv[b],
                            dsem[b]).wait()
      pltpu.async_copy(rows[b], acc.at[dstv[b]], ssem[b], add=True).wait()

      @pl.when(ci + NBUF < NCHUNKS)
      def _prefetch_idx():
        load_idx(ci + NBUF, b)

      @pl.when(ci + NBUF - 1 < NCHUNKS)
      def _prefetch_gather():
        gather(ci + NBUF - 1, bo)

    @pl.loop(0, NMAIN, step=NBUF)
    def _group(ci0):
      for b in range(NBUF):
        _do_chunk(ci0 + b, b, (b + NBUF - 1) % NBUF)

    for ci in range(NMAIN, NCHUNKS):
      _do_chunk(ci, ci % NBUF, (ci + NBUF - 1) % NBUF)

    plsc.subcore_barrier()
    pltpu.sync_copy(acc.at[pl.ds(sid * OROWS, OROWS)],
                    out_hbm.at[cid, pl.ds(sid * OROWS, OROWS)])

    @pl.when(sid == NS - 1)
    def _tail():
      pltpu.sync_copy(acc.at[pl.ds(OROWS * NS, OTAIL)],
                      out_hbm.at[cid, pl.ds(OROWS * NS, OTAIL)])

  return k(x, edge_index)


def _tc_add(partial):
  def body(a_ref, b_ref, o_ref):
    o_ref[...] = a_ref[0] + b_ref[0]

  blk = 1000
  return pl.pallas_call(
      body,
      out_shape=jax.ShapeDtypeStruct((N_NODES, D_FEAT), jnp.float32),
      grid=(N_NODES // blk,),
      in_specs=[
          pl.BlockSpec((1, blk, D_FEAT), lambda i: (0, i, 0)),
          pl.BlockSpec((1, blk, D_FEAT), lambda i: (1, i, 0)),
      ],
      out_specs=pl.BlockSpec((blk, D_FEAT), lambda i: (i, 0)),
  )(partial, partial)


@jax.jit
def kernel(x, edge_index):
  partial = _sc_scatter_gather(x, edge_index.reshape(2 * N_EDGES))
  return _tc_add(partial)
